# baseline (speedup 1.0000x reference)
import functools

import jax
import jax.numpy as jnp
from jax import lax
from jax.experimental import pallas as pl
from jax.experimental.pallas import tpu as pltpu
from jax.experimental.pallas import tpu_sc as plsc

NUM_WORKERS = 32
K = 8


@functools.partial(jax.jit, static_argnames=("batch", "seq", "dim"))
def _sc_gather(ids, table, *, batch, seq, dim):
    rows_per_w = batch // NUM_WORKERS
    n_groups = rows_per_w // K

    mesh = plsc.VectorSubcoreMesh(core_axis_name="c", subcore_axis_name="s")

    @functools.partial(
        pl.kernel,
        out_type=jax.ShapeDtypeStruct((batch * seq, 2, dim), jnp.float32),
        mesh=mesh,
        scratch_types=[
            pltpu.VMEM((K, seq), jnp.int32),
            pltpu.VMEM((K, seq, dim), jnp.float32),
            pltpu.SemaphoreType.DMA((K,)),
            pltpu.SemaphoreType.DMA((K,)),
            pltpu.SemaphoreType.DMA((K,)),
        ],
        compiler_params=pltpu.CompilerParams(use_tc_tiling_on_sc=False),
    )
    def k(ids_hbm, table_hbm, out_hbm, idx_v, rows_v, isem, gsem, ssem):
        wid = lax.axis_index("s") * 2 + lax.axis_index("c")
        b0 = wid * rows_per_w

        def body(g, carry):
            r0 = b0 + g * K
            for b in range(K):
                pltpu.async_copy(ids_hbm.at[r0 + b], idx_v.at[b], isem.at[b])
            for b in range(K):
                pltpu.make_async_copy(
                    ids_hbm.at[r0 + b], idx_v.at[b], isem.at[b]
                ).wait()
                pltpu.async_copy(
                    table_hbm.at[idx_v.at[b]], rows_v.at[b], gsem.at[b]
                )
            for b in range(K):
                pltpu.make_async_copy(
                    table_hbm.at[idx_v.at[b]], rows_v.at[b], gsem.at[b]
                ).wait()
                pltpu.async_copy(
                    rows_v.at[b],
                    out_hbm.at[pl.ds((r0 + b) * seq, seq), 0, :],
                    ssem.at[b],
                )
            for b in range(K):
                pltpu.make_async_copy(
                    rows_v.at[b],
                    out_hbm.at[pl.ds((r0 + b) * seq, seq), 0, :],
                    ssem.at[b],
                ).wait()
            return carry

        lax.fori_loop(0, n_groups, body, 0)

    return k(ids, table)


def kernel(input_ids, token_embed):
    batch, seq = input_ids.shape
    dim = token_embed.shape[1]
    out2 = _sc_gather(input_ids, token_embed, batch=batch, seq=seq, dim=dim)
    return out2.reshape(batch, seq, 2 * dim)[..., :dim]
